# baseline (speedup 1.0000x reference)
import jax
import jax.numpy as jnp
from jax.experimental import pallas as pl
from jax.experimental.pallas import tpu as pltpu

N_EMB = 8192
DIM = 32
N_TOK = 8192
TILE = 256
GRID = N_TOK // TILE


def _vq_body(x_ref, cb_ref, loss_ref, quant_ref, soft_ref, perp_ref, idx_ref,
             cbn_ref, counts_ref, lsum_ref):
    i = pl.program_id(0)

    @pl.when(i == 0)
    def _init():
        cb = cb_ref[...]
        n = jnp.sqrt(jnp.sum(cb * cb, axis=1, keepdims=True))
        cbn_ref[...] = cb / jnp.maximum(n, 1e-12)
        counts_ref[...] = jnp.zeros_like(counts_ref)
        lsum_ref[0, 0] = 0.0

    x = x_ref[...]
    xn = x / jnp.maximum(jnp.sqrt(jnp.sum(x * x, axis=1, keepdims=True)), 1e-12)
    cbn = cbn_ref[...]
    logits = jax.lax.dot_general(xn, cbn, (((1,), (1,)), ((), ())),
                                 preferred_element_type=jnp.float32)
    d = 2.0 - 2.0 * logits
    dmin = jnp.min(d, axis=1, keepdims=True)
    e = jnp.exp((dmin - d) * 10.0)
    soft_ref[...] = e * (1.0 / jnp.sum(e, axis=1, keepdims=True))

    col = jax.lax.broadcasted_iota(jnp.int32, (TILE, N_EMB), 1)
    idx = jnp.min(jnp.where(d == dmin, col, N_EMB), axis=1).astype(jnp.int32)
    idx_ref[0, 0, :] = idx

    onehot = (col == idx[:, None]).astype(jnp.float32)
    q = jax.lax.dot_general(onehot, cbn, (((1,), (0,)), ((), ())),
                            preferred_element_type=jnp.float32)
    quant_ref[...] = q
    diff = q - xn
    lsum_ref[0, 0] += jnp.sum(diff * diff)
    counts_ref[...] += jnp.sum(onehot, axis=0, keepdims=True)

    @pl.when(i == GRID - 1)
    def _fin():
        loss_ref[...] = jnp.reshape(0.25 * lsum_ref[0, 0] / (N_TOK * DIM), (1, 1))
        avg = counts_ref[...] / N_TOK
        perp_ref[...] = jnp.reshape(jnp.exp(-jnp.sum(avg * jnp.log(avg + 1e-10))), (1, 1))


def kernel(inputs, codebook):
    flat = inputs.reshape(-1, DIM)
    loss, quant, soft, perp, idx = pl.pallas_call(
        _vq_body,
        grid=(GRID,),
        in_specs=[
            pl.BlockSpec((TILE, DIM), lambda i: (i, 0)),
            pl.BlockSpec((N_EMB, DIM), lambda i: (0, 0)),
        ],
        out_specs=[
            pl.BlockSpec((1, 1), lambda i: (0, 0)),
            pl.BlockSpec((TILE, DIM), lambda i: (i, 0)),
            pl.BlockSpec((TILE, N_EMB), lambda i: (i, 0)),
            pl.BlockSpec((1, 1), lambda i: (0, 0)),
            pl.BlockSpec((1, 1, TILE), lambda i: (i, 0, 0)),
        ],
        out_shape=[
            jax.ShapeDtypeStruct((1, 1), jnp.float32),
            jax.ShapeDtypeStruct((N_TOK, DIM), jnp.float32),
            jax.ShapeDtypeStruct((N_TOK, N_EMB), jnp.float32),
            jax.ShapeDtypeStruct((1, 1), jnp.float32),
            jax.ShapeDtypeStruct((GRID, 1, TILE), jnp.int32),
        ],
        scratch_shapes=[
            pltpu.VMEM((N_EMB, DIM), jnp.float32),
            pltpu.VMEM((1, N_EMB), jnp.float32),
            pltpu.SMEM((1, 1), jnp.float32),
        ],
    )(flat, codebook)
    return (loss[0, 0], quant.reshape(inputs.shape), soft, perp[0, 0],
            idx.reshape(-1, 1))


# speedup vs baseline: 1.1931x; 1.1931x over previous
import jax
import jax.numpy as jnp
from jax.experimental import pallas as pl
from jax.experimental.pallas import tpu as pltpu

N_EMB = 8192
DIM = 32
N_TOK = 8192
TILE = 256
GRID = N_TOK // TILE


def _vq_body(x_ref, cb_ref, loss_ref, quant_ref, soft_ref, perp_ref, idx_ref,
             cbn_ref, counts_ref, lsum_ref):
    i = pl.program_id(0)

    @pl.when(i == 0)
    def _init():
        cb = cb_ref[...]
        n = jnp.sqrt(jnp.sum(cb * cb, axis=1, keepdims=True))
        cbn_ref[...] = cb / jnp.maximum(n, 1e-12)
        counts_ref[...] = jnp.zeros_like(counts_ref)
        lsum_ref[0, 0] = 0.0

    x = x_ref[...]
    xn = x / jnp.maximum(jnp.sqrt(jnp.sum(x * x, axis=1, keepdims=True)), 1e-12)
    cbn = cbn_ref[...]
    logits = jax.lax.dot_general(xn, cbn, (((1,), (1,)), ((), ())),
                                 preferred_element_type=jnp.float32)
    d = 2.0 - 2.0 * logits
    dmin = jnp.min(d, axis=1, keepdims=True)
    e = jnp.exp((dmin - d) * 10.0)
    soft_ref[...] = e * (1.0 / jnp.sum(e, axis=1, keepdims=True))

    idx = jnp.argmin(d, axis=1).astype(jnp.int32)
    idx_ref[0, 0, :] = idx

    col = jax.lax.broadcasted_iota(jnp.int32, (TILE, N_EMB), 1)
    onehot = (col == idx[:, None]).astype(jnp.float32)
    q = jax.lax.dot_general(onehot, cbn, (((1,), (0,)), ((), ())),
                            preferred_element_type=jnp.float32)
    quant_ref[...] = q
    diff = q - xn
    lsum_ref[0, 0] += jnp.sum(diff * diff)
    counts_ref[...] += jnp.sum(onehot, axis=0, keepdims=True)

    @pl.when(i == GRID - 1)
    def _fin():
        loss_ref[...] = jnp.reshape(0.25 * lsum_ref[0, 0] / (N_TOK * DIM), (1, 1))
        avg = counts_ref[...] / N_TOK
        perp_ref[...] = jnp.reshape(jnp.exp(-jnp.sum(avg * jnp.log(avg + 1e-10))), (1, 1))


def kernel(inputs, codebook):
    flat = inputs.reshape(-1, DIM)
    loss, quant, soft, perp, idx = pl.pallas_call(
        _vq_body,
        grid=(GRID,),
        in_specs=[
            pl.BlockSpec((TILE, DIM), lambda i: (i, 0)),
            pl.BlockSpec((N_EMB, DIM), lambda i: (0, 0)),
        ],
        out_specs=[
            pl.BlockSpec((1, 1), lambda i: (0, 0)),
            pl.BlockSpec((TILE, DIM), lambda i: (i, 0)),
            pl.BlockSpec((TILE, N_EMB), lambda i: (i, 0)),
            pl.BlockSpec((1, 1), lambda i: (0, 0)),
            pl.BlockSpec((1, 1, TILE), lambda i: (i, 0, 0)),
        ],
        out_shape=[
            jax.ShapeDtypeStruct((1, 1), jnp.float32),
            jax.ShapeDtypeStruct((N_TOK, DIM), jnp.float32),
            jax.ShapeDtypeStruct((N_TOK, N_EMB), jnp.float32),
            jax.ShapeDtypeStruct((1, 1), jnp.float32),
            jax.ShapeDtypeStruct((GRID, 1, TILE), jnp.int32),
        ],
        scratch_shapes=[
            pltpu.VMEM((N_EMB, DIM), jnp.float32),
            pltpu.VMEM((1, N_EMB), jnp.float32),
            pltpu.SMEM((1, 1), jnp.float32),
        ],
    )(flat, codebook)
    return (loss[0, 0], quant.reshape(inputs.shape), soft, perp[0, 0],
            idx.reshape(-1, 1))


# speedup vs baseline: 1.2107x; 1.0147x over previous
import jax
import jax.numpy as jnp
from jax.experimental import pallas as pl
from jax.experimental.pallas import tpu as pltpu

N_EMB = 8192
DIM = 32
N_TOK = 8192
TILE = 256
GRID = N_TOK // TILE


def _prep_body(cb_ref, cbn_ref):
    cb = cb_ref[...]
    n = jnp.sqrt(jnp.sum(cb * cb, axis=1, keepdims=True))
    cbn_ref[...] = cb / jnp.maximum(n, 1e-12)


def _vq_body(x_ref, cb_ref, loss_ref, quant_ref, soft_ref, perp_ref, idx_ref,
             counts_ref, lsum_ref):
    i = pl.program_id(0)

    @pl.when(i == 0)
    def _init():
        counts_ref[...] = jnp.zeros_like(counts_ref)
        lsum_ref[0, 0] = 0.0

    x = x_ref[...]
    xn = x / jnp.maximum(jnp.sqrt(jnp.sum(x * x, axis=1, keepdims=True)), 1e-12)
    cbn = cb_ref[...]
    logits = jax.lax.dot_general(xn, cbn, (((1,), (1,)), ((), ())),
                                 preferred_element_type=jnp.float32)
    d = 2.0 - 2.0 * logits
    t = -d / 0.1
    tmax = jnp.max(t, axis=1, keepdims=True)
    e = jnp.exp(t - tmax)
    soft_ref[...] = e / jnp.sum(e, axis=1, keepdims=True)

    idx = jnp.argmin(d, axis=1).astype(jnp.int32)
    idx_ref[0, 0, :] = idx

    col = jax.lax.broadcasted_iota(jnp.int32, (TILE, N_EMB), 1)
    onehot = (col == idx[:, None]).astype(jnp.float32)
    q = jax.lax.dot_general(onehot, cbn, (((1,), (0,)), ((), ())),
                            preferred_element_type=jnp.float32)
    quant_ref[...] = q
    diff = q - xn
    lsum_ref[0, 0] += jnp.sum(diff * diff)
    counts_ref[...] += jnp.sum(onehot, axis=0, keepdims=True)

    @pl.when(i == GRID - 1)
    def _fin():
        loss_ref[...] = jnp.reshape(0.25 * lsum_ref[0, 0] / (N_TOK * DIM), (1, 1))
        avg = counts_ref[...] / N_TOK
        perp_ref[...] = jnp.reshape(jnp.exp(-jnp.sum(avg * jnp.log(avg + 1e-10))), (1, 1))


def kernel(inputs, codebook):
    flat = inputs.reshape(-1, DIM)
    cbn = pl.pallas_call(
        _prep_body,
        out_shape=jax.ShapeDtypeStruct((N_EMB, DIM), jnp.float32),
    )(codebook)
    loss, quant, soft, perp, idx = pl.pallas_call(
        _vq_body,
        grid=(GRID,),
        in_specs=[
            pl.BlockSpec((TILE, DIM), lambda i: (i, 0)),
            pl.BlockSpec((N_EMB, DIM), lambda i: (0, 0)),
        ],
        out_specs=[
            pl.BlockSpec((1, 1), lambda i: (0, 0)),
            pl.BlockSpec((TILE, DIM), lambda i: (i, 0)),
            pl.BlockSpec((TILE, N_EMB), lambda i: (i, 0)),
            pl.BlockSpec((1, 1), lambda i: (0, 0)),
            pl.BlockSpec((1, 1, TILE), lambda i: (i, 0, 0)),
        ],
        out_shape=[
            jax.ShapeDtypeStruct((1, 1), jnp.float32),
            jax.ShapeDtypeStruct((N_TOK, DIM), jnp.float32),
            jax.ShapeDtypeStruct((N_TOK, N_EMB), jnp.float32),
            jax.ShapeDtypeStruct((1, 1), jnp.float32),
            jax.ShapeDtypeStruct((GRID, 1, TILE), jnp.int32),
        ],
        scratch_shapes=[
            pltpu.VMEM((1, N_EMB), jnp.float32),
            pltpu.SMEM((1, 1), jnp.float32),
        ],
    )(flat, cbn)
    return (loss[0, 0], quant.reshape(inputs.shape), soft, perp[0, 0],
            idx.reshape(-1, 1))


# speedup vs baseline: 1.2118x; 1.0009x over previous
import jax
import jax.numpy as jnp
from jax.experimental import pallas as pl
from jax.experimental.pallas import tpu as pltpu

N_EMB = 8192
DIM = 32
N_TOK = 8192
TILE = 512
GRID = N_TOK // TILE


def _prep_body(cb_ref, cbn_ref):
    cb = cb_ref[...]
    n = jnp.sqrt(jnp.sum(cb * cb, axis=1, keepdims=True))
    cbn_ref[...] = cb / jnp.maximum(n, 1e-12)


def _vq_body(x_ref, cb_ref, loss_ref, quant_ref, soft_ref, perp_ref, idx_ref,
             counts_ref, lsum_ref):
    i = pl.program_id(0)

    @pl.when(i == 0)
    def _init():
        counts_ref[...] = jnp.zeros_like(counts_ref)
        lsum_ref[0, 0] = 0.0

    x = x_ref[...]
    xn = x / jnp.maximum(jnp.sqrt(jnp.sum(x * x, axis=1, keepdims=True)), 1e-12)
    cbn = cb_ref[...]
    logits = jax.lax.dot_general(xn, cbn, (((1,), (1,)), ((), ())),
                                 preferred_element_type=jnp.float32)
    d = 2.0 - 2.0 * logits
    t = -d / 0.1
    tmax = jnp.max(t, axis=1, keepdims=True)
    e = jnp.exp(t - tmax)
    soft_ref[...] = e / jnp.sum(e, axis=1, keepdims=True)

    idx = jnp.argmin(d, axis=1).astype(jnp.int32)
    idx_ref[0, 0, :] = idx

    col = jax.lax.broadcasted_iota(jnp.int32, (TILE, N_EMB), 1)
    onehot = (col == idx[:, None]).astype(jnp.float32)
    q = jax.lax.dot_general(onehot, cbn, (((1,), (0,)), ((), ())),
                            preferred_element_type=jnp.float32)
    quant_ref[...] = q
    diff = q - xn
    lsum_ref[0, 0] += jnp.sum(diff * diff)
    counts_ref[...] += jnp.sum(onehot, axis=0, keepdims=True)

    @pl.when(i == GRID - 1)
    def _fin():
        loss_ref[...] = jnp.reshape(0.25 * lsum_ref[0, 0] / (N_TOK * DIM), (1, 1))
        avg = counts_ref[...] / N_TOK
        perp_ref[...] = jnp.reshape(jnp.exp(-jnp.sum(avg * jnp.log(avg + 1e-10))), (1, 1))


def kernel(inputs, codebook):
    flat = inputs.reshape(-1, DIM)
    cbn = pl.pallas_call(
        _prep_body,
        out_shape=jax.ShapeDtypeStruct((N_EMB, DIM), jnp.float32),
    )(codebook)
    loss, quant, soft, perp, idx = pl.pallas_call(
        _vq_body,
        grid=(GRID,),
        in_specs=[
            pl.BlockSpec((TILE, DIM), lambda i: (i, 0)),
            pl.BlockSpec((N_EMB, DIM), lambda i: (0, 0)),
        ],
        out_specs=[
            pl.BlockSpec((1, 1), lambda i: (0, 0)),
            pl.BlockSpec((TILE, DIM), lambda i: (i, 0)),
            pl.BlockSpec((TILE, N_EMB), lambda i: (i, 0)),
            pl.BlockSpec((1, 1), lambda i: (0, 0)),
            pl.BlockSpec((1, 1, TILE), lambda i: (i, 0, 0)),
        ],
        out_shape=[
            jax.ShapeDtypeStruct((1, 1), jnp.float32),
            jax.ShapeDtypeStruct((N_TOK, DIM), jnp.float32),
            jax.ShapeDtypeStruct((N_TOK, N_EMB), jnp.float32),
            jax.ShapeDtypeStruct((1, 1), jnp.float32),
            jax.ShapeDtypeStruct((GRID, 1, TILE), jnp.int32),
        ],
        scratch_shapes=[
            pltpu.VMEM((1, N_EMB), jnp.float32),
            pltpu.SMEM((1, 1), jnp.float32),
        ],
    )(flat, cbn)
    return (loss[0, 0], quant.reshape(inputs.shape), soft, perp[0, 0],
            idx.reshape(-1, 1))


# speedup vs baseline: 1.4581x; 1.2032x over previous
import functools

import jax
import jax.numpy as jnp
from jax.experimental import pallas as pl
from jax.experimental.pallas import tpu as pltpu
from jax.experimental.pallas import tpu_sc as plsc

N_EMB = 8192
DIM = 32
N_TOK = 8192
TILE = 512
GRID = N_TOK // TILE

_NC = 2
_NS = 16
_NW = _NC * _NS
_BPW = N_TOK // _NW


_PAD = 128


def _prep_body(cb_ref, cbn_ref, cbnp_ref):
    cb = cb_ref[...]
    n = jnp.sqrt(jnp.sum(cb * cb, axis=1, keepdims=True))
    cbn = cb / jnp.maximum(n, 1e-12)
    cbn_ref[...] = cbn
    cbnp_ref[...] = jnp.concatenate(
        [cbn, jnp.zeros((N_EMB, _PAD - DIM), jnp.float32)], axis=1)


def _vq_body(x_ref, cb_ref, loss_ref, soft_ref, idx_ref, lsum_ref):
    i = pl.program_id(0)

    @pl.when(i == 0)
    def _init():
        lsum_ref[0, 0] = 0.0

    x = x_ref[...]
    xn = x / jnp.maximum(jnp.sqrt(jnp.sum(x * x, axis=1, keepdims=True)), 1e-12)
    cbn = cb_ref[...]
    logits = jax.lax.dot_general(xn, cbn, (((1,), (1,)), ((), ())),
                                 preferred_element_type=jnp.float32)
    d = 2.0 - 2.0 * logits
    t = -d / 0.1
    tmax = jnp.max(t, axis=1, keepdims=True)
    e = jnp.exp(t - tmax)
    soft_ref[...] = e / jnp.sum(e, axis=1, keepdims=True)

    idx = jnp.argmin(d, axis=1).astype(jnp.int32)
    idx_ref[0, 0, :] = idx
    lsum_ref[0, 0] += jnp.sum(tmax)

    @pl.when(i == GRID - 1)
    def _fin():
        loss_ref[...] = jnp.reshape(
            0.25 * (-0.1) * lsum_ref[0, 0] / (N_TOK * DIM), (1, 1))


def _sc_body(cbn_hbm, idx_hbm, quant_hbm, counts_hbm, idx_v, rows_v, table_v,
             sem):
    c = jax.lax.axis_index("c")
    s = jax.lax.axis_index("s")
    wid = s * _NC + c
    base = wid * _BPW
    pltpu.sync_copy(idx_hbm.at[pl.ds(base, _BPW)], idx_v)
    pltpu.async_copy(cbn_hbm.at[idx_v], rows_v, sem).wait()
    pltpu.sync_copy(rows_v, quant_hbm.at[pl.ds(base, _BPW)])

    def _zero(j, carry):
        table_v[pl.ds(j * 16, 16)] = jnp.zeros((16,), jnp.float32)
        return carry

    jax.lax.fori_loop(0, N_EMB // 16, _zero, 0)

    def _hist(j, carry):
        idx16 = idx_v[pl.ds(j * 16, 16)]
        plsc.addupdate_scatter(table_v, [idx16], jnp.ones((16,), jnp.float32))
        return carry

    jax.lax.fori_loop(0, _BPW // 16, _hist, 0)
    pltpu.sync_copy(table_v, counts_hbm.at[wid])


_sc_gather_hist = functools.partial(
    pl.kernel,
    mesh=plsc.VectorSubcoreMesh(core_axis_name="c", subcore_axis_name="s"),
    out_type=[
        jax.ShapeDtypeStruct((N_TOK, _PAD), jnp.float32),
        jax.ShapeDtypeStruct((_NW, N_EMB), jnp.float32),
    ],
    scratch_types=[
        pltpu.VMEM((_BPW,), jnp.int32),
        pltpu.VMEM((_BPW, _PAD), jnp.float32),
        pltpu.VMEM((N_EMB,), jnp.float32),
        pltpu.SemaphoreType.DMA,
    ],
    compiler_params=pltpu.CompilerParams(needs_layout_passes=False),
)(_sc_body)


def _perp_body(cnt_ref, perp_ref):
    counts = jnp.sum(cnt_ref[...], axis=0, keepdims=True)
    avg = counts / N_TOK
    perp_ref[...] = jnp.reshape(
        jnp.exp(-jnp.sum(avg * jnp.log(avg + 1e-10))), (1, 1))


def kernel(inputs, codebook):
    flat = inputs.reshape(-1, DIM)
    cbn, cbnp = pl.pallas_call(
        _prep_body,
        out_shape=[
            jax.ShapeDtypeStruct((N_EMB, DIM), jnp.float32),
            jax.ShapeDtypeStruct((N_EMB, _PAD), jnp.float32),
        ],
    )(codebook)
    loss, soft, idx = pl.pallas_call(
        _vq_body,
        grid=(GRID,),
        in_specs=[
            pl.BlockSpec((TILE, DIM), lambda i: (i, 0)),
            pl.BlockSpec((N_EMB, DIM), lambda i: (0, 0)),
        ],
        out_specs=[
            pl.BlockSpec((1, 1), lambda i: (0, 0)),
            pl.BlockSpec((TILE, N_EMB), lambda i: (i, 0)),
            pl.BlockSpec((1, 1, TILE), lambda i: (i, 0, 0)),
        ],
        out_shape=[
            jax.ShapeDtypeStruct((1, 1), jnp.float32),
            jax.ShapeDtypeStruct((N_TOK, N_EMB), jnp.float32),
            jax.ShapeDtypeStruct((GRID, 1, TILE), jnp.int32),
        ],
        scratch_shapes=[
            pltpu.SMEM((1, 1), jnp.float32),
        ],
    )(flat, cbn)
    idx_flat = idx.reshape(-1)
    quant_pad, counts = _sc_gather_hist(cbnp, idx_flat)
    quant = quant_pad[:, :DIM]
    perp = pl.pallas_call(
        _perp_body,
        out_shape=jax.ShapeDtypeStruct((1, 1), jnp.float32),
    )(counts)
    return (loss[0, 0], quant.reshape(inputs.shape), soft, perp[0, 0],
            idx_flat[:, None])
